# baseline (speedup 1.0000x reference)
import functools

import jax
import jax.numpy as jnp
from jax import lax
from jax.experimental import pallas as pl
from jax.experimental.pallas import tpu as pltpu
from jax.experimental.pallas import tpu_sc as plsc

BATCH = 16384
D0 = 4096
M = 64
H = 64
BB = 512

NC = 2
NS = 16
NW = NC * NS
RPW = BATCH // NW
CH = 128
NCH = RPW // CH


def _top2_onehot(e, n):
    it = lax.broadcasted_iota(jnp.int32, e.shape, 0)
    m1 = jnp.max(e, axis=0, keepdims=True)
    i1 = jnp.min(jnp.where(e == m1, it, n), axis=0, keepdims=True)
    em = jnp.where(it == i1, -jnp.inf, e)
    m2 = jnp.max(em, axis=0, keepdims=True)
    i2 = jnp.min(jnp.where(em == m2, it, n), axis=0, keepdims=True)
    s1 = (it == i1).astype(jnp.float32)
    s2 = (it == i2).astype(jnp.float32)
    return (jnp.concatenate([s1, s2], axis=1),
            jnp.concatenate([i1, i2], axis=1))


def _route_body(e0_ref, e1_ref, e2_ref, r0_ref, b2f0_ref, r1_ref, b2f1_ref,
                cols_ref, g0_ref, c0_ref, g1_ref, c1_ref):
    f32 = jnp.float32
    _, i0 = _top2_onehot(e0_ref[...], D0)
    cols_ref[...] = jnp.broadcast_to(i0, (8, 2 * M))
    s1, _ = _top2_onehot(e1_ref[...], M)
    g0_ref[...] = jnp.dot(r0_ref[...], s1, preferred_element_type=f32)
    c0_ref[...] = jnp.broadcast_to(
        jnp.dot(b2f0_ref[0:1, :], s1, preferred_element_type=f32), (8, 2 * M))
    s2, _ = _top2_onehot(e2_ref[...], M)
    g1_ref[...] = jnp.dot(r1_ref[...], s2, preferred_element_type=f32)
    c1_ref[...] = jnp.broadcast_to(
        jnp.dot(b2f1_ref[0:1, :], s2, preferred_element_type=f32), (8, 2))


def _sc_gather(xflat, cols):
    mesh = plsc.VectorSubcoreMesh(core_axis_name="c", subcore_axis_name="s")
    CW = CH * 2 * M

    @functools.partial(
        pl.kernel, mesh=mesh,
        out_type=jax.ShapeDtypeStruct((BATCH * 2 * M,), jnp.float32),
        scratch_types=(
            [pltpu.VMEM((2 * M,), jnp.int32)]
            + [pltpu.VMEM((CW,), jnp.int32) for _ in range(2)]
            + [pltpu.VMEM((CW,), jnp.float32) for _ in range(2)]
            + [pltpu.SemaphoreType.DMA for _ in range(2)]
        ),
    )
    def gk(x_hbm, cols_hbm, out_hbm, cols_v, idx0, idx1, buf0, buf1,
           sem0, sem1):
        idx_v, buf_v, sems = (idx0, idx1), (buf0, buf1), (sem0, sem1)
        wid = lax.axis_index("s") * NC + lax.axis_index("c")
        row0 = wid * RPW
        pltpu.sync_copy(cols_hbm, cols_v)

        def fill_and_fire(ch):
            s = ch % 2
            base = (row0 + ch * CH) * D0

            def fill(r, _, s=s, base=base):
                off = base + r * D0
                for j in range(2 * M // 16):
                    idx_v[s][pl.ds(r * 2 * M + 16 * j, 16)] = (
                        cols_v[pl.ds(16 * j, 16)] + off)
                return 0

            lax.fori_loop(0, CH, fill, 0)
            return pltpu.async_copy(x_hbm.at[idx_v[s]], buf_v[s], sems[s])

        cps = {0: fill_and_fire(0)}
        for ch in range(NCH):
            if ch + 1 < NCH:
                cps[ch + 1] = fill_and_fire(ch + 1)
            cps[ch].wait()
            pltpu.sync_copy(buf_v[ch % 2],
                            out_hbm.at[pl.ds((row0 + ch * CH) * 2 * M, CW)])

    return gk(xflat, cols)


def _main_body(u_ref, p0_ref, b1f0_ref, g0_ref, c0_ref,
               p1_ref, b1f1_ref, g1_ref, c1_ref, o_ref):
    f32 = jnp.float32
    a0 = jnp.maximum(jnp.dot(u_ref[...], p0_ref[...], preferred_element_type=f32)
                     + b1f0_ref[0:1, :], 0.0)
    u1 = jnp.dot(a0, g0_ref[...], preferred_element_type=f32) + c0_ref[0:1, :]
    a1 = jnp.maximum(jnp.dot(u1, p1_ref[...], preferred_element_type=f32)
                     + b1f1_ref[0:1, :], 0.0)
    o_ref[...] = jax.nn.sigmoid(
        jnp.dot(a1, g1_ref[...], preferred_element_type=f32) + c1_ref[0:1, :])


def _block_weights(W1, b1, W2, b2):
    eye = jnp.eye(M, dtype=jnp.float32)
    P = jnp.einsum('mn,mhi->imnh', eye, W1).reshape(2 * M, M * H)
    b1f = jnp.broadcast_to(b1.reshape(1, M * H), (8, M * H))
    R = jnp.einsum('mn,mh->mhn', eye, W2[:, 0, :]).reshape(M * H, M)
    b2f = jnp.broadcast_to(b2.reshape(1, M), (8, M))
    return P, b1f, R, b2f


def kernel(x, task_id, emb0, emb1, emb2,
           mlp0_W1, mlp0_b1, mlp0_W2, mlp0_b2,
           mlp1_W1, mlp1_b1, mlp1_W2, mlp1_b2):
    e0 = lax.dynamic_index_in_dim(emb0, task_id, 0, keepdims=False)
    e1 = lax.dynamic_index_in_dim(emb1, task_id, 0, keepdims=False)
    e2 = lax.dynamic_index_in_dim(emb2, task_id, 0, keepdims=False)

    P0, b1f0, R0, b2f0 = _block_weights(mlp0_W1, mlp0_b1, mlp0_W2, mlp0_b2)
    P1, b1f1, R1, b2f1 = _block_weights(mlp1_W1, mlp1_b1, mlp1_W2, mlp1_b2)

    cols8, G0, c0, G1, c1 = pl.pallas_call(
        _route_body,
        out_shape=(
            jax.ShapeDtypeStruct((8, 2 * M), jnp.int32),
            jax.ShapeDtypeStruct((M * H, 2 * M), jnp.float32),
            jax.ShapeDtypeStruct((8, 2 * M), jnp.float32),
            jax.ShapeDtypeStruct((M * H, 2), jnp.float32),
            jax.ShapeDtypeStruct((8, 2), jnp.float32),
        ),
    )(e0, e1, e2, R0, b2f0, R1, b2f1)

    u = _sc_gather(x.reshape(-1), cols8[0]).reshape(BATCH, 2 * M)

    nblk = BATCH // BB
    full = lambda shape: pl.BlockSpec(shape, lambda i: (0, 0))
    out = pl.pallas_call(
        _main_body,
        grid=(nblk,),
        in_specs=[
            pl.BlockSpec((BB, 2 * M), lambda i: (i, 0)),
            full((2 * M, M * H)), full((8, M * H)),
            full((M * H, 2 * M)), full((8, 2 * M)),
            full((2 * M, M * H)), full((8, M * H)),
            full((M * H, 2)), full((8, 2)),
        ],
        out_specs=pl.BlockSpec((BB, 2), lambda i: (i, 0)),
        out_shape=jax.ShapeDtypeStruct((BATCH, 2), jnp.float32),
    )(u, P0, b1f0, G0, c0, P1, b1f1, G1, c1)
    return out


# speedup vs baseline: 1.5487x; 1.5487x over previous
import jax
import jax.numpy as jnp
from jax import lax
from jax.experimental import pallas as pl

BATCH = 16384
D0 = 4096
M = 64
H = 64
BB = 512


def _top2_onehot(e, n):
    it = lax.broadcasted_iota(jnp.int32, e.shape, 0)
    m1 = jnp.max(e, axis=0, keepdims=True)
    i1 = jnp.min(jnp.where(e == m1, it, n), axis=0, keepdims=True)
    em = jnp.where(it == i1, -jnp.inf, e)
    m2 = jnp.max(em, axis=0, keepdims=True)
    i2 = jnp.min(jnp.where(em == m2, it, n), axis=0, keepdims=True)
    s1 = (it == i1).astype(jnp.float32)
    s2 = (it == i2).astype(jnp.float32)
    return (jnp.concatenate([s1, s2], axis=1),
            jnp.concatenate([i1, i2], axis=1))


def _route_body(e0_ref, e1_ref, e2_ref, r0_ref, b2f0_ref, r1_ref, b2f1_ref,
                s0_ref, g0_ref, c0_ref, g1_ref, c1_ref):
    f32 = jnp.float32
    s0, _ = _top2_onehot(e0_ref[...], D0)
    s0_ref[...] = s0
    s1, _ = _top2_onehot(e1_ref[...], M)
    g0_ref[...] = jnp.dot(r0_ref[...], s1,
                          preferred_element_type=f32).astype(jnp.bfloat16)
    c0_ref[...] = jnp.broadcast_to(
        jnp.dot(b2f0_ref[0:1, :], s1, preferred_element_type=f32), (8, 2 * M))
    s2, _ = _top2_onehot(e2_ref[...], M)
    g1_ref[...] = jnp.dot(r1_ref[...], s2, preferred_element_type=f32)
    c1_ref[...] = jnp.broadcast_to(
        jnp.dot(b2f1_ref[0:1, :], s2, preferred_element_type=f32), (8, 2))


def _main_body(x_ref, s0_ref, p0_ref, b1f0_ref, g0_ref, c0_ref,
               p1_ref, b1f1_ref, g1_ref, c1_ref, o_ref):
    f32 = jnp.float32
    bf16 = jnp.bfloat16
    u0 = jnp.dot(x_ref[...], s0_ref[...], preferred_element_type=f32)
    a0 = jnp.maximum(
        jnp.dot(u0.astype(bf16), p0_ref[...], preferred_element_type=f32)
        + b1f0_ref[0:1, :], 0.0)
    u1 = jnp.dot(a0.astype(bf16), g0_ref[...],
                 preferred_element_type=f32) + c0_ref[0:1, :]
    a1 = jnp.maximum(
        jnp.dot(u1.astype(bf16), p1_ref[...], preferred_element_type=f32)
        + b1f1_ref[0:1, :], 0.0)
    o_ref[...] = jax.nn.sigmoid(
        jnp.dot(a1, g1_ref[...], preferred_element_type=f32) + c1_ref[0:1, :])


def _block_weights(W1, b1, W2, b2):
    eye = jnp.eye(M, dtype=jnp.float32)
    P = jnp.einsum('mn,mhi->imnh', eye, W1).reshape(2 * M, M * H)
    b1f = jnp.broadcast_to(b1.reshape(1, M * H), (8, M * H))
    R = jnp.einsum('mn,mh->mhn', eye, W2[:, 0, :]).reshape(M * H, M)
    b2f = jnp.broadcast_to(b2.reshape(1, M), (8, M))
    return P, b1f, R, b2f


def kernel(x, task_id, emb0, emb1, emb2,
           mlp0_W1, mlp0_b1, mlp0_W2, mlp0_b2,
           mlp1_W1, mlp1_b1, mlp1_W2, mlp1_b2):
    e0 = lax.dynamic_index_in_dim(emb0, task_id, 0, keepdims=False)
    e1 = lax.dynamic_index_in_dim(emb1, task_id, 0, keepdims=False)
    e2 = lax.dynamic_index_in_dim(emb2, task_id, 0, keepdims=False)

    P0, b1f0, R0, b2f0 = _block_weights(mlp0_W1, mlp0_b1, mlp0_W2, mlp0_b2)
    P1, b1f1, R1, b2f1 = _block_weights(mlp1_W1, mlp1_b1, mlp1_W2, mlp1_b2)

    s0, G0, c0, G1, c1 = pl.pallas_call(
        _route_body,
        out_shape=(
            jax.ShapeDtypeStruct((D0, 2 * M), jnp.float32),
            jax.ShapeDtypeStruct((M * H, 2 * M), jnp.bfloat16),
            jax.ShapeDtypeStruct((8, 2 * M), jnp.float32),
            jax.ShapeDtypeStruct((M * H, 2), jnp.float32),
            jax.ShapeDtypeStruct((8, 2), jnp.float32),
        ),
    )(e0, e1, e2, R0, b2f0, R1, b2f1)

    P0b = P0.astype(jnp.bfloat16)
    P1b = P1.astype(jnp.bfloat16)

    nblk = BATCH // BB
    full = lambda shape: pl.BlockSpec(shape, lambda i: (0, 0))
    out = pl.pallas_call(
        _main_body,
        grid=(nblk,),
        in_specs=[
            pl.BlockSpec((BB, D0), lambda i: (i, 0)),
            full((D0, 2 * M)),
            full((2 * M, M * H)), full((8, M * H)),
            full((M * H, 2 * M)), full((8, 2 * M)),
            full((2 * M, M * H)), full((8, M * H)),
            full((M * H, 2)), full((8, 2)),
        ],
        out_specs=pl.BlockSpec((BB, 2), lambda i: (i, 0)),
        out_shape=jax.ShapeDtypeStruct((BATCH, 2), jnp.float32),
    )(x, s0, P0b, b1f0, G0, c0, P1b, b1f1, G1, c1)
    return out


# speedup vs baseline: 1.8596x; 1.2007x over previous
import jax
import jax.numpy as jnp
from jax import lax
from jax.experimental import pallas as pl

BATCH = 16384
D0 = 4096
M = 64
H = 64
BB = 1024


def _top2_onehot(e, n):
    it = lax.broadcasted_iota(jnp.int32, e.shape, 0)
    m1 = jnp.max(e, axis=0, keepdims=True)
    i1 = jnp.min(jnp.where(e == m1, it, n), axis=0, keepdims=True)
    em = jnp.where(it == i1, -jnp.inf, e)
    m2 = jnp.max(em, axis=0, keepdims=True)
    i2 = jnp.min(jnp.where(em == m2, it, n), axis=0, keepdims=True)
    s1 = (it == i1).astype(jnp.float32)
    s2 = (it == i2).astype(jnp.float32)
    return (jnp.concatenate([s1, s2], axis=1),
            jnp.concatenate([i1, i2], axis=1))


def _route_body(e0_ref, e1_ref, e2_ref, r0_ref, b2f0_ref, r1_ref, b2f1_ref,
                s0_ref, g0_ref, c0_ref, g1_ref, c1_ref):
    f32 = jnp.float32
    s0, _ = _top2_onehot(e0_ref[...], D0)
    s0_ref[...] = s0
    s1, _ = _top2_onehot(e1_ref[...], M)
    g0_ref[...] = jnp.dot(r0_ref[...], s1,
                          preferred_element_type=f32).astype(jnp.bfloat16)
    c0_ref[...] = jnp.broadcast_to(
        jnp.dot(b2f0_ref[0:1, :], s1, preferred_element_type=f32), (8, 2 * M))
    s2, _ = _top2_onehot(e2_ref[...], M)
    g1_ref[...] = jnp.dot(r1_ref[...], s2, preferred_element_type=f32)
    c1_ref[...] = jnp.broadcast_to(
        jnp.dot(b2f1_ref[0:1, :], s2, preferred_element_type=f32), (8, 2))


def _main_body(x_ref, s0_ref, p0_ref, b1f0_ref, g0_ref, c0_ref,
               p1_ref, b1f1_ref, g1_ref, c1_ref, o_ref):
    f32 = jnp.float32
    bf16 = jnp.bfloat16
    hb = BB // 4
    for h in range(4):
        rows = pl.ds(h * hb, hb)
        u0 = jnp.dot(x_ref[rows, :], s0_ref[...], preferred_element_type=f32)
        a0 = jnp.maximum(
            jnp.dot(u0.astype(bf16), p0_ref[...], preferred_element_type=f32)
            + b1f0_ref[0:1, :], 0.0)
        u1 = jnp.dot(a0.astype(bf16), g0_ref[...],
                     preferred_element_type=f32) + c0_ref[0:1, :]
        a1 = jnp.maximum(
            jnp.dot(u1.astype(bf16), p1_ref[...], preferred_element_type=f32)
            + b1f1_ref[0:1, :], 0.0)
        o_ref[rows, :] = jax.nn.sigmoid(
            jnp.dot(a1, g1_ref[...], preferred_element_type=f32)
            + c1_ref[0:1, :])


def _block_weights(W1, b1, W2, b2):
    eye = jnp.eye(M, dtype=jnp.float32)
    P = jnp.einsum('mn,mhi->imnh', eye, W1).reshape(2 * M, M * H)
    b1f = jnp.broadcast_to(b1.reshape(1, M * H), (8, M * H))
    R = jnp.einsum('mn,mh->mhn', eye, W2[:, 0, :]).reshape(M * H, M)
    b2f = jnp.broadcast_to(b2.reshape(1, M), (8, M))
    return P, b1f, R, b2f


def kernel(x, task_id, emb0, emb1, emb2,
           mlp0_W1, mlp0_b1, mlp0_W2, mlp0_b2,
           mlp1_W1, mlp1_b1, mlp1_W2, mlp1_b2):
    e0 = lax.dynamic_index_in_dim(emb0, task_id, 0, keepdims=False)
    e1 = lax.dynamic_index_in_dim(emb1, task_id, 0, keepdims=False)
    e2 = lax.dynamic_index_in_dim(emb2, task_id, 0, keepdims=False)

    P0, b1f0, R0, b2f0 = _block_weights(mlp0_W1, mlp0_b1, mlp0_W2, mlp0_b2)
    P1, b1f1, R1, b2f1 = _block_weights(mlp1_W1, mlp1_b1, mlp1_W2, mlp1_b2)

    s0, G0, c0, G1, c1 = pl.pallas_call(
        _route_body,
        out_shape=(
            jax.ShapeDtypeStruct((D0, 2 * M), jnp.float32),
            jax.ShapeDtypeStruct((M * H, 2 * M), jnp.bfloat16),
            jax.ShapeDtypeStruct((8, 2 * M), jnp.float32),
            jax.ShapeDtypeStruct((M * H, 2), jnp.float32),
            jax.ShapeDtypeStruct((8, 2), jnp.float32),
        ),
    )(e0, e1, e2, R0, b2f0, R1, b2f1)

    P0b = P0.astype(jnp.bfloat16)
    P1b = P1.astype(jnp.bfloat16)

    nblk = BATCH // BB
    full = lambda shape: pl.BlockSpec(shape, lambda i: (0, 0))
    out = pl.pallas_call(
        _main_body,
        grid=(nblk,),
        in_specs=[
            pl.BlockSpec((BB, D0), lambda i: (i, 0)),
            full((D0, 2 * M)),
            full((2 * M, M * H)), full((8, M * H)),
            full((M * H, 2 * M)), full((8, 2 * M)),
            full((2 * M, M * H)), full((8, M * H)),
            full((M * H, 2)), full((8, 2)),
        ],
        out_specs=pl.BlockSpec((BB, 2), lambda i: (i, 0)),
        out_shape=jax.ShapeDtypeStruct((BATCH, 2), jnp.float32),
    )(x, s0, P0b, b1f0, G0, c0, P1b, b1f1, G1, c1)
    return out
